# baseline (speedup 1.0000x reference)
import functools

import jax
import jax.numpy as jnp
from jax import lax
from jax.experimental import pallas as pl
from jax.experimental.pallas import tpu as pltpu
from jax.experimental.pallas import tpu_sc as plsc

_VOCAB = 1000000
_SEQ = 2048
_EMBD = 64
_BATCH = 16
_N = _BATCH * _SEQ
_NC, _NS = 2, 16
_NW = _NC * _NS
_PER_W = _N // _NW
_CHUNK = 128
_HALF = _PER_W // 2
_CPH = _HALF // _CHUNK


def _body(tok_hbm, table_hbm, pos_hbm, out_hbm, idx_v, rows_v, pos_v, sem):
    wid = lax.axis_index("s") * _NC + lax.axis_index("c")
    base = wid * _PER_W
    pos_base = base % _SEQ

    pltpu.sync_copy(tok_hbm.at[pl.ds(wid * (_PER_W // _CHUNK), _PER_W // _CHUNK)],
                    idx_v)

    for h in range(2):
        row0 = h * _HALF
        copies = []
        for c in range(_CPH):
            cp = pltpu.make_async_copy(
                table_hbm.at[idx_v.at[h * _CPH + c]],
                rows_v.at[pl.ds(c * _CHUNK, _CHUNK)],
                sem,
            )
            cp.start()
            copies.append(cp)
        pltpu.sync_copy(pos_hbm.at[pl.ds(pos_base + row0, _HALF)], pos_v)
        for cp in copies:
            cp.wait()

        def add_row(i, carry):
            for j in range(_EMBD // 16):
                sl = pl.ds(j * 16, 16)
                rows_v[i, sl] = rows_v[i, sl] + pos_v[i, sl]
            return carry

        lax.fori_loop(0, _HALF, add_row, 0)
        pltpu.sync_copy(rows_v, out_hbm.at[pl.ds(base + row0, _HALF)])


@jax.jit
def _embed(tok2d, table, pos):
    mesh = plsc.VectorSubcoreMesh(core_axis_name="c", subcore_axis_name="s",
                                  num_cores=_NC, num_subcores=_NS)
    run = pl.kernel(
        _body,
        out_type=jax.ShapeDtypeStruct((_N, _EMBD), jnp.float32),
        mesh=mesh,
        scratch_types=[
            pltpu.VMEM((_PER_W // _CHUNK, _CHUNK), jnp.int32),
            pltpu.VMEM((_HALF, _EMBD), jnp.float32),
            pltpu.VMEM((_HALF, _EMBD), jnp.float32),
            pltpu.SemaphoreType.DMA,
        ],
        compiler_params=pltpu.CompilerParams(use_tc_tiling_on_sc=False),
    )
    return run(tok2d, table, pos)


def kernel(token_ids, token_table, pos_table):
    tok2d = token_ids.astype(jnp.int32).reshape(_N // _CHUNK, _CHUNK)
    out = _embed(tok2d, token_table, pos_table)
    return out.reshape(_BATCH, _SEQ, _EMBD)
